# baseline (speedup 1.0000x reference)
import functools

import jax
import jax.numpy as jnp
from jax.experimental import pallas as pl
from jax.experimental.pallas import tpu as pltpu

LANE = 128


def _round_up(x, m):
    return ((x + m - 1) // m) * m


def _rnn_kernel(x_ref,
                wih0_ref,
                whh0_ref,
                bih0_ref,
                bhh0_ref,
                wih1_ref,
                whh1_ref,
                bih1_ref,
                bhh1_ref,
                wp_ref,
                bp_ref,
                y_ref,
                xs_ref,
                z0_ref,
                h0_ref,
                h1_ref,
                wih0b_ref,
                whh0b_ref,
                wih1b_ref,
                whh1b_ref,
                wpb_ref,
                *, n_streams):
    b_idx = pl.program_id(0)
    tc = pl.program_id(1)
    n_tc = pl.num_programs(1)
    Bt, Tc, I = x_ref.shape
    H = h0_ref.shape[-1]
    Sw = Bt // n_streams

    @pl.when((b_idx == 0) & (tc == 0))
    def _():
        wih0b_ref[...] = wih0_ref[...].astype(jnp.bfloat16)
        whh0b_ref[...] = whh0_ref[...].astype(jnp.bfloat16)
        wih1b_ref[...] = wih1_ref[...].astype(jnp.bfloat16)
        whh1b_ref[...] = whh1_ref[...].astype(jnp.bfloat16)
        wpb_ref[...] = wp_ref[...].astype(jnp.bfloat16)

    piece = min(8, Tc)
    for t0 in range(0, Tc, piece):
        pe = min(piece, Tc - t0)
        xp = jnp.swapaxes(x_ref[:, t0:t0 + pe, :].astype(jnp.bfloat16),
                          0, 1).reshape(pe * Bt, I)
        xs_ref[pl.ds(t0 * Bt, pe * Bt), :] = xp
    b0 = bih0_ref[...] + bhh0_ref[...]
    z0_ref[...] = (jnp.dot(xs_ref[...], wih0b_ref[...],
                           preferred_element_type=jnp.float32) + b0)

    @pl.when(tc == 0)
    def _():
        h0_ref[...] = jnp.zeros_like(h0_ref)
        h1_ref[...] = jnp.zeros_like(h1_ref)

    b1 = jnp.broadcast_to(bih1_ref[...] + bhh1_ref[...], (Sw, H))

    def z0_at(t, s):
        base = pl.multiple_of(t * Bt + s * Sw, Sw)
        return z0_ref[pl.ds(base, Sw), :]

    def srows(s):
        return pl.ds(s * Sw, Sw)

    h0s = []
    h1s = []
    for s in range(n_streams):
        h0c = h0_ref[srows(s), :]
        z = z0_at(0, s) + jnp.dot(h0c, whh0b_ref[...],
                                  preferred_element_type=jnp.float32)
        h0s.append(jnp.tanh(z).astype(jnp.bfloat16))
        h1s.append(h1_ref[srows(s), :])

    def body(t, carry):
        h0, h1 = carry
        new0, new1 = [], []
        for s in range(n_streams):
            z1 = (jnp.dot(h0[s], wih1b_ref[...],
                          preferred_element_type=jnp.float32)
                  + jnp.dot(h1[s], whh1b_ref[...],
                            preferred_element_type=jnp.float32) + b1)
            new1.append(jnp.tanh(z1).astype(jnp.bfloat16))
            z0 = z0_at(t, s) + jnp.dot(h0[s], whh0b_ref[...],
                                       preferred_element_type=jnp.float32)
            new0.append(jnp.tanh(z0).astype(jnp.bfloat16))
        return (tuple(new0), tuple(new1))

    carry = (tuple(h0s), tuple(h1s))
    for t in range(1, Tc):
        carry = body(t, carry)
    h0s, h1s = carry

    for s in range(n_streams):
        z1 = (jnp.dot(h0s[s], wih1b_ref[...], preferred_element_type=jnp.float32)
              + jnp.dot(h1s[s], whh1b_ref[...], preferred_element_type=jnp.float32)
              + b1)
        h1_last = jnp.tanh(z1)
        h0_ref[srows(s), :] = h0s[s]
        h1_ref[srows(s), :] = h1_last.astype(jnp.bfloat16)

        @pl.when(tc == n_tc - 1)
        def _():
            y = (jnp.dot(h1_last.astype(jnp.bfloat16), wpb_ref[...],
                         preferred_element_type=jnp.float32) + bp_ref[...])
            y_ref[srows(s), :] = y


def _aligned_rnn(x, wih0, whh0, bih0, bhh0, wih1, whh1, bih1, bhh1, wp, bp):
    B, T, I = x.shape
    H = whh0.shape[0]
    C = wp.shape[1]

    b_tile = min(256, B)
    n_bt = B // b_tile
    n_streams = 1

    t_chunk = 1
    for c in range(min(T, 32), 0, -1):
        if T % c == 0:
            t_chunk = c
            break
    n_tc = T // t_chunk

    const_spec = lambda a: pl.BlockSpec(a.shape, lambda b, t: (0,) * a.ndim)

    grid_spec = pltpu.PrefetchScalarGridSpec(
        num_scalar_prefetch=0,
        grid=(n_bt, n_tc),
        in_specs=[
            pl.BlockSpec((b_tile, t_chunk, I), lambda b, t: (b, t, 0)),
            const_spec(wih0), const_spec(whh0),
            const_spec(bih0), const_spec(bhh0),
            const_spec(wih1), const_spec(whh1),
            const_spec(bih1), const_spec(bhh1),
            const_spec(wp), const_spec(bp),
        ],
        out_specs=pl.BlockSpec((b_tile, C), lambda b, t: (b, 0)),
        scratch_shapes=[
            pltpu.VMEM((t_chunk * b_tile, I), jnp.bfloat16),
            pltpu.VMEM((t_chunk * b_tile, H), jnp.float32),
            pltpu.VMEM((b_tile, H), jnp.bfloat16),
            pltpu.VMEM((b_tile, H), jnp.bfloat16),
            pltpu.VMEM((I, H), jnp.bfloat16),
            pltpu.VMEM((H, H), jnp.bfloat16),
            pltpu.VMEM((H, H), jnp.bfloat16),
            pltpu.VMEM((H, H), jnp.bfloat16),
            pltpu.VMEM((H, C), jnp.bfloat16),
        ],
    )

    return pl.pallas_call(
        functools.partial(_rnn_kernel, n_streams=n_streams),
        out_shape=jax.ShapeDtypeStruct((B, C), jnp.float32),
        grid_spec=grid_spec,
        compiler_params=pltpu.CompilerParams(
            dimension_semantics=("arbitrary", "arbitrary"),
            vmem_limit_bytes=64 * 1024 * 1024),
    )(x, wih0, whh0, bih0, bhh0, wih1, whh1, bih1, bhh1, wp, bp)


def kernel(x, wih0, whh0, bih0, bhh0, wih1, whh1, bih1, bhh1, wp, bp):
    B, T, I = x.shape
    H = whh0.shape[0]
    C = wp.shape[1]

    if H % LANE == 0 and C % LANE == 0 and I % LANE == 0 and B % 8 == 0:
        return _aligned_rnn(x, wih0, whh0, bih0, bhh0, wih1, whh1,
                            bih1, bhh1, wp, bp)

    Hp = _round_up(H, LANE)
    Cp = _round_up(C, LANE)
    Ip = _round_up(I, LANE)
    Bp = _round_up(B, 8)
    pad2 = lambda a, r, c: jnp.pad(a, ((0, r - a.shape[0]), (0, c - a.shape[1])))
    xp = jnp.pad(x, ((0, Bp - B), (0, 0), (0, Ip - I)))
    y = _aligned_rnn(
        xp,
        pad2(wih0, Ip, Hp), pad2(whh0, Hp, Hp),
        pad2(bih0, 1, Hp), pad2(bhh0, 1, Hp),
        pad2(wih1, Hp, Hp), pad2(whh1, Hp, Hp),
        pad2(bih1, 1, Hp), pad2(bhh1, 1, Hp),
        pad2(wp, Hp, Cp), pad2(bp, 1, Cp),
    )
    return y[:B, :C]


# speedup vs baseline: 1.3527x; 1.3527x over previous
import functools

import jax
import jax.numpy as jnp
from jax.experimental import pallas as pl
from jax.experimental.pallas import tpu as pltpu

LANE = 128


def _round_up(x, m):
    return ((x + m - 1) // m) * m


def _rnn_kernel(x_ref,
                wih0_ref,
                whh0_ref,
                bih0_ref,
                bhh0_ref,
                wih1_ref,
                whh1_ref,
                bih1_ref,
                bhh1_ref,
                wp_ref,
                bp_ref,
                y_ref,
                xs_ref,
                z0_ref,
                h0_ref,
                h1_ref,
                wih0b_ref,
                wcb_ref,
                whh1b_ref,
                wpb_ref):
    b_idx = pl.program_id(0)
    tc = pl.program_id(1)
    n_tc = pl.num_programs(1)
    Bt, Tc, I = x_ref.shape
    H = h0_ref.shape[-1]

    @pl.when((b_idx == 0) & (tc == 0))
    def _():
        wih0b_ref[...] = wih0_ref[...].astype(jnp.bfloat16)
        wcb_ref[:, :H] = whh0_ref[...].astype(jnp.bfloat16)
        wcb_ref[:, H:] = wih1_ref[...].astype(jnp.bfloat16)
        whh1b_ref[...] = whh1_ref[...].astype(jnp.bfloat16)
        wpb_ref[...] = wp_ref[...].astype(jnp.bfloat16)

    piece = min(8, Tc)
    for t0 in range(0, Tc, piece):
        pe = min(piece, Tc - t0)
        xp = jnp.swapaxes(x_ref[:, t0:t0 + pe, :].astype(jnp.bfloat16),
                          0, 1).reshape(pe * Bt, I)
        xs_ref[pl.ds(t0 * Bt, pe * Bt), :] = xp
    b0 = bih0_ref[...] + bhh0_ref[...]
    z0_ref[...] = (jnp.dot(xs_ref[...], wih0b_ref[...],
                           preferred_element_type=jnp.float32) + b0)

    @pl.when(tc == 0)
    def _():
        h0_ref[...] = jnp.zeros_like(h0_ref)
        h1_ref[...] = jnp.zeros_like(h1_ref)

    b1 = jnp.broadcast_to(bih1_ref[...] + bhh1_ref[...], (Bt, H))

    def z0_at(t):
        return z0_ref[pl.ds(pl.multiple_of(t * Bt, Bt), Bt), :]

    def body(t, carry):
        h0, h1 = carry
        zc = jnp.dot(h0, wcb_ref[...], preferred_element_type=jnp.float32)
        z1 = zc[:, H:] + jnp.dot(h1, whh1b_ref[...],
                                 preferred_element_type=jnp.float32) + b1
        h1n = jnp.tanh(z1).astype(jnp.bfloat16)
        z0 = z0_at(t) + zc[:, :H]
        h0n = jnp.tanh(z0).astype(jnp.bfloat16)
        return (h0n, h1n)

    def chain(t_lo):
        carry = (h0_ref[...], h1_ref[...])
        for t in range(t_lo, Tc):
            carry = body(t, carry)
        h0_ref[...], h1_ref[...] = carry

    @pl.when(tc == 0)
    def _():
        h0_ref[...] = jnp.tanh(z0_at(0)).astype(jnp.bfloat16)
        chain(1)

    @pl.when(tc != 0)
    def _():
        chain(0)

    @pl.when(tc == n_tc - 1)
    def _():
        h0 = h0_ref[...]
        h1 = h1_ref[...]
        z1 = (jnp.dot(h0, wcb_ref[:, H:], preferred_element_type=jnp.float32)
              + jnp.dot(h1, whh1b_ref[...], preferred_element_type=jnp.float32)
              + b1)
        h1_last = jnp.tanh(z1).astype(jnp.bfloat16)
        y_ref[...] = (jnp.dot(h1_last, wpb_ref[...],
                              preferred_element_type=jnp.float32) + bp_ref[...])


def _aligned_rnn(x, wih0, whh0, bih0, bhh0, wih1, whh1, bih1, bhh1, wp, bp):
    B, T, I = x.shape
    H = whh0.shape[0]
    C = wp.shape[1]

    b_tile = min(256, B)
    n_bt = B // b_tile

    t_chunk = 1
    for c in range(min(T, 32), 0, -1):
        if T % c == 0:
            t_chunk = c
            break
    n_tc = T // t_chunk

    const_spec = lambda a: pl.BlockSpec(a.shape, lambda b, t: (0,) * a.ndim)

    grid_spec = pltpu.PrefetchScalarGridSpec(
        num_scalar_prefetch=0,
        grid=(n_bt, n_tc),
        in_specs=[
            pl.BlockSpec((b_tile, t_chunk, I), lambda b, t: (b, t, 0)),
            const_spec(wih0), const_spec(whh0),
            const_spec(bih0), const_spec(bhh0),
            const_spec(wih1), const_spec(whh1),
            const_spec(bih1), const_spec(bhh1),
            const_spec(wp), const_spec(bp),
        ],
        out_specs=pl.BlockSpec((b_tile, C), lambda b, t: (b, 0)),
        scratch_shapes=[
            pltpu.VMEM((t_chunk * b_tile, I), jnp.bfloat16),
            pltpu.VMEM((t_chunk * b_tile, H), jnp.float32),
            pltpu.VMEM((b_tile, H), jnp.bfloat16),
            pltpu.VMEM((b_tile, H), jnp.bfloat16),
            pltpu.VMEM((I, H), jnp.bfloat16),
            pltpu.VMEM((H, 2 * H), jnp.bfloat16),
            pltpu.VMEM((H, H), jnp.bfloat16),
            pltpu.VMEM((H, C), jnp.bfloat16),
        ],
    )

    return pl.pallas_call(
        _rnn_kernel,
        out_shape=jax.ShapeDtypeStruct((B, C), jnp.float32),
        grid_spec=grid_spec,
        compiler_params=pltpu.CompilerParams(
            dimension_semantics=("arbitrary", "arbitrary"),
            vmem_limit_bytes=64 * 1024 * 1024),
    )(x, wih0, whh0, bih0, bhh0, wih1, whh1, bih1, bhh1, wp, bp)


def kernel(x, wih0, whh0, bih0, bhh0, wih1, whh1, bih1, bhh1, wp, bp):
    B, T, I = x.shape
    H = whh0.shape[0]
    C = wp.shape[1]

    if H % LANE == 0 and C % LANE == 0 and I % LANE == 0 and B % 8 == 0:
        return _aligned_rnn(x, wih0, whh0, bih0, bhh0, wih1, whh1,
                            bih1, bhh1, wp, bp)

    Hp = _round_up(H, LANE)
    Cp = _round_up(C, LANE)
    Ip = _round_up(I, LANE)
    Bp = _round_up(B, 8)
    pad2 = lambda a, r, c: jnp.pad(a, ((0, r - a.shape[0]), (0, c - a.shape[1])))
    xp = jnp.pad(x, ((0, Bp - B), (0, 0), (0, Ip - I)))
    y = _aligned_rnn(
        xp,
        pad2(wih0, Ip, Hp), pad2(whh0, Hp, Hp),
        pad2(bih0, 1, Hp), pad2(bhh0, 1, Hp),
        pad2(wih1, Hp, Hp), pad2(whh1, Hp, Hp),
        pad2(bih1, 1, Hp), pad2(bhh1, 1, Hp),
        pad2(wp, Hp, Cp), pad2(bp, 1, Cp),
    )
    return y[:B, :C]


# speedup vs baseline: 1.3687x; 1.0118x over previous
import functools

import jax
import jax.numpy as jnp
from jax.experimental import pallas as pl
from jax.experimental.pallas import tpu as pltpu

LANE = 128


def _round_up(x, m):
    return ((x + m - 1) // m) * m


def _rnn_kernel(x_ref,
                wih0_ref,
                whh0_ref,
                bih0_ref,
                bhh0_ref,
                wih1_ref,
                whh1_ref,
                bih1_ref,
                bhh1_ref,
                wp_ref,
                bp_ref,
                y_ref,
                xs_ref,
                z0_ref,
                h0_ref,
                h1_ref,
                wih0b_ref,
                wcb_ref,
                whh1b_ref,
                wpb_ref):
    b_idx = pl.program_id(0)
    tc = pl.program_id(1)
    n_tc = pl.num_programs(1)
    Bt, Tc, I = x_ref.shape
    H = h0_ref.shape[-1]

    @pl.when((b_idx == 0) & (tc == 0))
    def _():
        wih0b_ref[...] = wih0_ref[...].astype(jnp.bfloat16)
        wcb_ref[:, :H] = whh0_ref[...].astype(jnp.bfloat16)
        wcb_ref[:, H:] = wih1_ref[...].astype(jnp.bfloat16)
        whh1b_ref[...] = whh1_ref[...].astype(jnp.bfloat16)
        wpb_ref[...] = wp_ref[...].astype(jnp.bfloat16)

    piece = min(8, Tc)
    for t0 in range(0, Tc, piece):
        pe = min(piece, Tc - t0)
        xp = jnp.swapaxes(x_ref[:, t0:t0 + pe, :].astype(jnp.bfloat16),
                          0, 1).reshape(pe * Bt, I)
        xs_ref[pl.ds(t0 * Bt, pe * Bt), :] = xp
    b0 = bih0_ref[...] + bhh0_ref[...]
    z0_ref[...] = (jnp.dot(xs_ref[...], wih0b_ref[...],
                           preferred_element_type=jnp.float32) + b0)

    @pl.when(tc == 0)
    def _():
        h0_ref[...] = jnp.zeros_like(h0_ref)
        h1_ref[...] = jnp.zeros_like(h1_ref)

    b1 = jnp.broadcast_to(bih1_ref[...] + bhh1_ref[...], (Bt, H))

    def z0_at(t):
        return z0_ref[pl.ds(pl.multiple_of(t * Bt, Bt), Bt), :]

    def body(t, carry):
        h0, h1 = carry
        zc = jnp.dot(h0, wcb_ref[...], preferred_element_type=jnp.float32)
        z1 = zc[:, H:] + jnp.dot(h1, whh1b_ref[...],
                                 preferred_element_type=jnp.float32) + b1
        h1n = jnp.tanh(z1).astype(jnp.bfloat16)
        z0 = z0_at(t) + zc[:, :H]
        h0n = jnp.tanh(z0).astype(jnp.bfloat16)
        return (h0n, h1n)

    def chain(t_lo):
        carry = (h0_ref[...], h1_ref[...])
        for t in range(t_lo, Tc):
            carry = body(t, carry)
        h0_ref[...], h1_ref[...] = carry

    @pl.when(tc == 0)
    def _():
        h0_ref[...] = jnp.tanh(z0_at(0)).astype(jnp.bfloat16)
        chain(1)

    @pl.when(tc != 0)
    def _():
        chain(0)

    @pl.when(tc == n_tc - 1)
    def _():
        h0 = h0_ref[...]
        h1 = h1_ref[...]
        z1 = (jnp.dot(h0, wcb_ref[:, H:], preferred_element_type=jnp.float32)
              + jnp.dot(h1, whh1b_ref[...], preferred_element_type=jnp.float32)
              + b1)
        h1_last = jnp.tanh(z1).astype(jnp.bfloat16)
        y_ref[...] = (jnp.dot(h1_last, wpb_ref[...],
                              preferred_element_type=jnp.float32) + bp_ref[...])


def _aligned_rnn(x, wih0, whh0, bih0, bhh0, wih1, whh1, bih1, bhh1, wp, bp):
    B, T, I = x.shape
    H = whh0.shape[0]
    C = wp.shape[1]

    b_tile = min(256, B)
    n_bt = B // b_tile

    t_chunk = 1
    for c in range(min(T, 16), 0, -1):
        if T % c == 0:
            t_chunk = c
            break
    n_tc = T // t_chunk

    const_spec = lambda a: pl.BlockSpec(a.shape, lambda b, t: (0,) * a.ndim)

    grid_spec = pltpu.PrefetchScalarGridSpec(
        num_scalar_prefetch=0,
        grid=(n_bt, n_tc),
        in_specs=[
            pl.BlockSpec((b_tile, t_chunk, I), lambda b, t: (b, t, 0)),
            const_spec(wih0), const_spec(whh0),
            const_spec(bih0), const_spec(bhh0),
            const_spec(wih1), const_spec(whh1),
            const_spec(bih1), const_spec(bhh1),
            const_spec(wp), const_spec(bp),
        ],
        out_specs=pl.BlockSpec((b_tile, C), lambda b, t: (b, 0)),
        scratch_shapes=[
            pltpu.VMEM((t_chunk * b_tile, I), jnp.bfloat16),
            pltpu.VMEM((t_chunk * b_tile, H), jnp.float32),
            pltpu.VMEM((b_tile, H), jnp.bfloat16),
            pltpu.VMEM((b_tile, H), jnp.bfloat16),
            pltpu.VMEM((I, H), jnp.bfloat16),
            pltpu.VMEM((H, 2 * H), jnp.bfloat16),
            pltpu.VMEM((H, H), jnp.bfloat16),
            pltpu.VMEM((H, C), jnp.bfloat16),
        ],
    )

    return pl.pallas_call(
        _rnn_kernel,
        out_shape=jax.ShapeDtypeStruct((B, C), jnp.float32),
        grid_spec=grid_spec,
        compiler_params=pltpu.CompilerParams(
            dimension_semantics=("arbitrary", "arbitrary"),
            vmem_limit_bytes=64 * 1024 * 1024),
    )(x, wih0, whh0, bih0, bhh0, wih1, whh1, bih1, bhh1, wp, bp)


def kernel(x, wih0, whh0, bih0, bhh0, wih1, whh1, bih1, bhh1, wp, bp):
    B, T, I = x.shape
    H = whh0.shape[0]
    C = wp.shape[1]

    if H % LANE == 0 and C % LANE == 0 and I % LANE == 0 and B % 8 == 0:
        return _aligned_rnn(x, wih0, whh0, bih0, bhh0, wih1, whh1,
                            bih1, bhh1, wp, bp)

    Hp = _round_up(H, LANE)
    Cp = _round_up(C, LANE)
    Ip = _round_up(I, LANE)
    Bp = _round_up(B, 8)
    pad2 = lambda a, r, c: jnp.pad(a, ((0, r - a.shape[0]), (0, c - a.shape[1])))
    xp = jnp.pad(x, ((0, Bp - B), (0, 0), (0, Ip - I)))
    y = _aligned_rnn(
        xp,
        pad2(wih0, Ip, Hp), pad2(whh0, Hp, Hp),
        pad2(bih0, 1, Hp), pad2(bhh0, 1, Hp),
        pad2(wih1, Hp, Hp), pad2(whh1, Hp, Hp),
        pad2(bih1, 1, Hp), pad2(bhh1, 1, Hp),
        pad2(wp, Hp, Cp), pad2(bp, 1, Cp),
    )
    return y[:B, :C]
